# baseline (speedup 1.0000x reference)
import math

import jax
import jax.numpy as jnp
import numpy as np
from jax import lax
from jax.experimental import pallas as pl
from jax.experimental.pallas import tpu as pltpu
from jax.experimental.pallas import tpu_sc as plsc

VOCAB = 1000000
EMB = 64
SEQ = 200
BATCH = 4096

NUM_CORES = 2
NUM_SUBCORES = 16
NUM_WORKERS = NUM_CORES * NUM_SUBCORES

CHUNK = 100
TOTAL = BATCH * SEQ
PER_W = TOTAL // NUM_WORKERS
NCHUNK = PER_W // CHUNK
SCALE = math.sqrt(float(EMB))


def _positional_encoding_np(max_len, d):
    position = np.arange(max_len)[:, None].astype(np.float32)
    div_term = np.exp(np.arange(0, d, 2).astype(np.float32) * (-np.log(10000.0) / d))
    pe = np.zeros((max_len, d), dtype=np.float32)
    pe[:, 0::2] = np.sin(position * div_term)
    pe[:, 1::2] = np.cos(position * div_term)
    return pe


_PE = _positional_encoding_np(SEQ, EMB)


def _sc_kernel(x_hbm, table_hbm, pe_hbm, out_hbm,
               idx_v, in_v, out_v, pe_v, sem_g, sem_s):
    wid = lax.axis_index("s") * NUM_CORES + lax.axis_index("c")
    rows_per_w = PER_W // CHUNK

    pltpu.sync_copy(pe_hbm, pe_v)
    pltpu.sync_copy(x_hbm.at[pl.ds(wid * rows_per_w, rows_per_w)], idx_v)

    base = wid * PER_W

    def chunk_body(c, _):
        pltpu.async_copy(table_hbm.at[idx_v.at[c]], in_v, sem_g).wait()
        s0 = lax.rem(c, 2) * CHUNK

        def row_body(r, _):
            for d in range(EMB // 16):
                sl = pl.ds(d * 16, 16)
                out_v[pl.ds(r * EMB + d * 16, 16)] = (
                    in_v[r, sl] * SCALE + pe_v[s0 + r, sl])
            return 0

        lax.fori_loop(0, CHUNK, row_body, 0, unroll=2)
        pltpu.async_copy(out_v,
                         out_hbm.at[pl.ds((base + c * CHUNK) * EMB, CHUNK * EMB)],
                         sem_s).wait()
        return 0

    lax.fori_loop(0, NCHUNK, chunk_body, 0)


@jax.jit
def kernel(x, table):
    xf = x.reshape(TOTAL // CHUNK, CHUNK)
    mesh = plsc.VectorSubcoreMesh(core_axis_name="c", subcore_axis_name="s")
    out = pl.kernel(
        _sc_kernel,
        out_type=jax.ShapeDtypeStruct((TOTAL * EMB,), jnp.float32),
        mesh=mesh,
        compiler_params=pltpu.CompilerParams(use_tc_tiling_on_sc=False),
        scratch_types=[
            pltpu.VMEM((PER_W // CHUNK, CHUNK), jnp.int32),
            pltpu.VMEM((CHUNK, EMB), jnp.float32),
            pltpu.VMEM((CHUNK * EMB,), jnp.float32),
            pltpu.VMEM((SEQ, EMB), jnp.float32),
            pltpu.SemaphoreType.DMA,
            pltpu.SemaphoreType.DMA,
        ],
    )(xf, table, jnp.asarray(_PE))
    return out.reshape(BATCH, SEQ, EMB)


# speedup vs baseline: 1.2175x; 1.2175x over previous
import functools
import math

import jax
import jax.numpy as jnp
import numpy as np
from jax import lax
from jax.experimental import pallas as pl
from jax.experimental.pallas import tpu as pltpu
from jax.experimental.pallas import tpu_sc as plsc

VOCAB = 1000000
EMB = 64
SEQ = 200
BATCH = 4096

NUM_CORES = 2
NUM_SUBCORES = 16
NUM_WORKERS = NUM_CORES * NUM_SUBCORES

CHUNK = 128
TOTAL = BATCH * SEQ
PER_W = TOTAL // NUM_WORKERS
NCHUNK = PER_W // CHUNK
NBUF = 4
SCALE = math.sqrt(float(EMB))
PE_ROWS = 2 * SEQ


def _positional_encoding_np(max_len, d):
    position = np.arange(max_len)[:, None].astype(np.float32)
    div_term = np.exp(np.arange(0, d, 2).astype(np.float32) * (-np.log(10000.0) / d))
    pe = np.zeros((max_len, d), dtype=np.float32)
    pe[:, 0::2] = np.sin(position * div_term)
    pe[:, 1::2] = np.cos(position * div_term)
    return pe


_PE2 = np.concatenate([_positional_encoding_np(SEQ, EMB)] * 2, axis=0).reshape(-1)


def _sc_kernel(x_hbm, table_hbm, pe_hbm, out_hbm,
               idx_v, in0, in1, in2, in3, ou0, ou1, ou2, ou3, pe_v,
               sg0, sg1, sg2, sg3, ss0, ss1, ss2, ss3):
    ins = (in0, in1, in2, in3)
    outs = (ou0, ou1, ou2, ou3)
    sgs = (sg0, sg1, sg2, sg3)
    sss = (ss0, ss1, ss2, ss3)

    wid = lax.axis_index("s") * NUM_CORES + lax.axis_index("c")
    pltpu.sync_copy(pe_hbm, pe_v)
    pltpu.sync_copy(x_hbm.at[pl.ds(wid * NCHUNK, NCHUNK)], idx_v)
    base = wid * PER_W

    def g_cp(c, b):
        return pltpu.make_async_copy(table_hbm.at[idx_v.at[c]], ins[b], sgs[b])

    def s_cp(c, b):
        dst = out_hbm.at[pl.ds((base + c * CHUNK) * EMB, CHUNK * EMB)]
        return pltpu.make_async_copy(outs[b], dst, sss[b])

    for b in range(NBUF):
        g_cp(b, b).start()

    def quad(g, _):
        for b in range(NBUF):
            c = NBUF * g + b
            g_cp(c, b).wait()

            @pl.when(g >= 1)
            def _():
                s_cp(c - NBUF, b).wait()

            p0 = lax.rem(c * CHUNK, SEQ) * EMB
            i_buf = ins[b]
            o_buf = outs[b]

            def row(r, _):
                for d in range(EMB // 16):
                    o_buf[pl.ds(r * EMB + d * 16, 16)] = (
                        i_buf[r, pl.ds(d * 16, 16)] * SCALE
                        + pe_v[pl.ds(p0 + r * EMB + d * 16, 16)])
                return 0

            lax.fori_loop(0, CHUNK, row, 0, unroll=2)

            @pl.when(g < NCHUNK // NBUF - 1)
            def _():
                g_cp(c + NBUF, b).start()

            s_cp(c, b).start()
        return 0

    lax.fori_loop(0, NCHUNK // NBUF, quad, 0)
    for b in range(NBUF):
        s_cp(NCHUNK - NBUF + b, b).wait()


def _impl(x, table):
    xf = x.reshape(TOTAL // CHUNK, CHUNK)
    mesh = plsc.VectorSubcoreMesh(core_axis_name="c", subcore_axis_name="s")
    out = pl.kernel(
        _sc_kernel,
        out_type=jax.ShapeDtypeStruct((TOTAL * EMB,), jnp.float32),
        mesh=mesh,
        compiler_params=pltpu.CompilerParams(use_tc_tiling_on_sc=False),
        scratch_types=(
            [pltpu.VMEM((NCHUNK, CHUNK), jnp.int32)]
            + [pltpu.VMEM((CHUNK, EMB), jnp.float32) for _ in range(NBUF)]
            + [pltpu.VMEM((CHUNK * EMB,), jnp.float32) for _ in range(NBUF)]
            + [pltpu.VMEM((PE_ROWS * EMB,), jnp.float32)]
            + [pltpu.SemaphoreType.DMA] * (2 * NBUF)
        ),
    )(xf, table, jnp.asarray(_PE2))
    return out.reshape(BATCH, SEQ, EMB)


def kernel(x, table):
    return _impl(x, table)
